# baseline (speedup 1.0000x reference)
import jax
import jax.numpy as jnp
from jax.experimental import pallas as pl
from jax.experimental.pallas import tpu as pltpu

B, L, A = 128, 512, 1024
CB = 8
NB = 3


def _matvec_bias_kernel(d_hbm, c_ref, o_ref, ring, s0, s1, s2):
    sems = (s0, s1, s2)
    n_chunks = B // CB

    def issue(ci):
        slot = ci % NB
        pltpu.make_async_copy(
            d_hbm.at[pl.ds(ci * CB, CB)], ring.at[slot], sems[slot]
        ).start()

    for ci in range(NB - 1):
        issue(ci)

    for ci in range(n_chunks):
        slot = ci % NB
        pltpu.make_async_copy(
            d_hbm.at[pl.ds(ci * CB, CB)], ring.at[slot], sems[slot]
        ).wait()
        if ci + NB - 1 < n_chunks:
            issue(ci + NB - 1)
        d = ring[slot]
        c = c_ref[pl.ds(ci * CB, CB), :, :]
        acc = jnp.sum(d * c[:, :, :A], axis=-1)
        o_ref[pl.ds(ci * CB, CB), :, :] = acc[:, None, :] + c[:, :, A:A + 1]


def kernel(dict, coef):
    out = pl.pallas_call(
        _matvec_bias_kernel,
        in_specs=[
            pl.BlockSpec(memory_space=pl.ANY),
            pl.BlockSpec((B, 1, A + 1), lambda: (0, 0, 0)),
        ],
        out_specs=pl.BlockSpec((B, 1, L), lambda: (0, 0, 0)),
        out_shape=jax.ShapeDtypeStruct((B, 1, L), jnp.float32),
        scratch_shapes=[
            pltpu.VMEM((NB, CB, L, A), jnp.float32),
            pltpu.SemaphoreType.DMA,
            pltpu.SemaphoreType.DMA,
            pltpu.SemaphoreType.DMA,
        ],
    )(dict, coef[:, None, :])
    return out.reshape(B, L, 1)


# speedup vs baseline: 1.0417x; 1.0417x over previous
import jax
import jax.numpy as jnp
from jax.experimental import pallas as pl
from jax.experimental.pallas import tpu as pltpu

B, L, A = 128, 512, 1024
CB = 2
NB = 4


def _matvec_bias_kernel(d_hbm, c_ref, o_ref, ring, s0, s1, s2, s3):
    sems = (s0, s1, s2, s3)
    n_chunks = B // CB

    def issue(ci):
        slot = ci % NB
        pltpu.make_async_copy(
            d_hbm.at[pl.ds(ci * CB, CB)], ring.at[slot], sems[slot]
        ).start()

    for ci in range(NB - 1):
        issue(ci)

    for ci in range(n_chunks):
        slot = ci % NB
        pltpu.make_async_copy(
            d_hbm.at[pl.ds(ci * CB, CB)], ring.at[slot], sems[slot]
        ).wait()
        if ci + NB - 1 < n_chunks:
            issue(ci + NB - 1)
        d = ring[slot]
        c = c_ref[pl.ds(ci * CB, CB), :, :]
        acc = jnp.sum(d * c[:, :, :A], axis=-1)
        o_ref[pl.ds(ci * CB, CB), :, :] = acc[:, None, :] + c[:, :, A:A + 1]


def kernel(dict, coef):
    out = pl.pallas_call(
        _matvec_bias_kernel,
        in_specs=[
            pl.BlockSpec(memory_space=pl.ANY),
            pl.BlockSpec((B, 1, A + 1), lambda: (0, 0, 0)),
        ],
        out_specs=pl.BlockSpec((B, 1, L), lambda: (0, 0, 0)),
        out_shape=jax.ShapeDtypeStruct((B, 1, L), jnp.float32),
        scratch_shapes=[
            pltpu.VMEM((NB, CB, L, A), jnp.float32),
            pltpu.SemaphoreType.DMA,
            pltpu.SemaphoreType.DMA,
            pltpu.SemaphoreType.DMA,
            pltpu.SemaphoreType.DMA,
        ],
    )(dict, coef[:, None, :])
    return out.reshape(B, L, 1)
